# baseline (speedup 1.0000x reference)
import functools

import jax
import jax.numpy as jnp
from jax import lax
from jax.experimental import pallas as pl
from jax.experimental.pallas import tpu as pltpu
from jax.experimental.pallas import tpu_sc as plsc

N_NODES = 10000
N_PAD = 10112
N_EDGES = 320000
D = 128

NC = 2
NS = 16
NW = NC * NS
CH = 64
NB = 4
NPH = 8
PH = 20
EPW = NPH * PH * CH
E_PAD = NW * EPW
DROP_ROW = N_NODES + 8
ROWS_PER_TILE = N_PAD // NS


def _sc_aggregate(x, src, dst):
    mesh = plsc.VectorSubcoreMesh(core_axis_name="c", subcore_axis_name="s")

    @functools.partial(
        pl.kernel,
        mesh=mesh,
        out_type=jax.ShapeDtypeStruct((NC, N_PAD, D), jnp.float32),
        scratch_types=(
            [pltpu.VMEM((PH * CH,), jnp.int32)] * 4 +
            [pltpu.VMEM((PH, CH), jnp.int32)] +
            [pltpu.VMEM((CH, D), jnp.float32)] * NB +
            [pltpu.VMEM_SHARED((N_PAD, D), jnp.float32)] +
            [pltpu.SemaphoreType.DMA] * (NB + 3)
        ),
    )
    def agg(x_hbm, src_hbm, dst_hbm, out_hbm,
            sidx0, didx0, sidx1, didx1, didx2d, *rest):
        rows = list(rest[:NB])
        acc = rest[NB]
        gsem = list(rest[NB + 1:2 * NB + 1])
        zsem, isem0, isem1 = rest[2 * NB + 1:]
        cid = lax.axis_index("c")
        sid = lax.axis_index("s")
        wid = sid * NC + cid
        rbase = sid * ROWS_PER_TILE

        zval = jnp.zeros((16,), jnp.float32)

        def zrow(i, carry):
            for j in range(D // 16):
                rows[0][i, pl.ds(j * 16, 16)] = zval
            return carry

        lax.fori_loop(0, CH, zrow, 0)
        zcopies = [pltpu.async_copy(rows[0], acc.at[pl.ds(rbase + k * CH, CH)], zsem)
                   for k in range(ROWS_PER_TILE // CH)]
        ztail = ROWS_PER_TILE % CH
        if ztail:
            zcopies.append(pltpu.async_copy(
                rows[0].at[pl.ds(0, ztail)],
                acc.at[pl.ds(rbase + (ROWS_PER_TILE // CH) * CH, ztail)], zsem))

        def repack(dsrc):
            def rp(k, carry):
                didx2d[k // (CH // 16), pl.ds((k % (CH // 16)) * 16, 16)] = (
                    dsrc[pl.ds(k * 16, 16)])
                return carry

            lax.fori_loop(0, PH * CH // 16, rp, 0)

        ebase = wid * EPW
        pblk = PH * CH
        idx_bufs = [(sidx0, didx0, isem0), (sidx1, didx1, isem1)]
        pltpu.sync_copy(src_hbm.at[pl.ds(ebase, pblk)], sidx0)
        pltpu.sync_copy(dst_hbm.at[pl.ds(ebase, pblk)], didx0)
        pending = {1: (
            pltpu.async_copy(src_hbm.at[pl.ds(ebase + pblk, pblk)], sidx1, isem1),
            pltpu.async_copy(dst_hbm.at[pl.ds(ebase + pblk, pblk)], didx1, isem1))}
        repack(didx0)
        for b in range(1, NB):
            pltpu.async_copy(
                x_hbm.at[sidx0.at[pl.ds(b * CH, CH)]], rows[b], gsem[b])
        for h in zcopies:
            h.wait()
        pltpu.async_copy(x_hbm.at[sidx0.at[pl.ds(0, CH)]], rows[0], gsem[0])
        plsc.subcore_barrier()

        for p in range(NPH):
            sidx, didx1d, _ = idx_bufs[p % 2]
            if 1 <= p and p + 1 < NPH:
                ns, nd, nsem = idx_bufs[(p + 1) % 2]
                off = ebase + (p + 1) * pblk
                pending[p + 1] = (
                    pltpu.async_copy(src_hbm.at[pl.ds(off, pblk)], ns, nsem),
                    pltpu.async_copy(dst_hbm.at[pl.ds(off, pblk)], nd, nsem))
            for h in pending.pop(p, ()):
                h.wait()
            if p > 0:
                repack(didx1d)
                for b in range(NB):
                    pltpu.async_copy(
                        x_hbm.at[sidx.at[pl.ds(b * CH, CH)]], rows[b], gsem[b])

            def body(j4, carry, sidx=sidx):
                a = NB * j4
                for b in range(NB):
                    pltpu.make_async_copy(
                        x_hbm.at[sidx.at[pl.ds((a + b) * CH, CH)]],
                        rows[b], gsem[b]).wait()
                    pltpu.sync_copy(rows[b], acc.at[didx2d.at[a + b]], add=True)

                    @pl.when(a + b + NB < PH)
                    def _(b=b, a=a, sidx=sidx):
                        pltpu.async_copy(
                            x_hbm.at[sidx.at[pl.ds((a + b + NB) * CH, CH)]],
                            rows[b], gsem[b])
                return carry

            lax.fori_loop(0, PH // NB, body, 0)
        plsc.subcore_barrier()

        pltpu.sync_copy(acc.at[pl.ds(rbase, ROWS_PER_TILE)],
                        out_hbm.at[cid, pl.ds(rbase, ROWS_PER_TILE)])

    return agg(x, src, dst)


def _tc_combine(partials, W, bias):
    BR = 1000

    def body(p_ref, w_ref, b_ref, o_ref):
        s = p_ref[0] + p_ref[1]
        o_ref[...] = lax.dot_general(
            s, w_ref[...], (((1,), (1,)), ((), ())),
            preferred_element_type=jnp.float32) + b_ref[...]

    return pl.pallas_call(
        body,
        grid=(N_NODES // BR,),
        in_specs=[
            pl.BlockSpec((NC, BR, D), lambda i: (0, i, 0)),
            pl.BlockSpec((D, D), lambda i: (0, 0)),
            pl.BlockSpec((1, D), lambda i: (0, 0)),
        ],
        out_specs=pl.BlockSpec((BR, D), lambda i: (i, 0)),
        out_shape=jax.ShapeDtypeStruct((N_NODES, D), jnp.float32),
    )(partials, W, bias.reshape(1, D))


def kernel(x, edge_index, W, bias):
    ei = edge_index.astype(jnp.int32)
    n_extra = E_PAD - N_EDGES
    pad_iota = jnp.arange(n_extra, dtype=jnp.int32)
    src = jnp.concatenate([ei[0], pad_iota % N_NODES])
    dst = jnp.concatenate([ei[1], N_NODES + pad_iota % (N_PAD - N_NODES)])
    partials = _sc_aggregate(x, src, dst)
    return _tc_combine(partials, W, bias)


# speedup vs baseline: 1.0391x; 1.0391x over previous
import functools

import jax
import jax.numpy as jnp
from jax import lax
from jax.experimental import pallas as pl
from jax.experimental.pallas import tpu as pltpu
from jax.experimental.pallas import tpu_sc as plsc

N_NODES = 10000
N_PAD = 10112
N_EDGES = 320000
D = 128

NC = 2
NS = 16
NW = NC * NS
CH = 64
NB = 4
NPH = 10
PH = 16
EPW = NPH * PH * CH
E_PAD = NW * EPW
ROWS_PER_TILE = N_PAD // NS


def _sc_aggregate(x, edges):
    mesh = plsc.VectorSubcoreMesh(core_axis_name="c", subcore_axis_name="s")

    @functools.partial(
        pl.kernel,
        mesh=mesh,
        out_type=jax.ShapeDtypeStruct((NC, N_PAD, D), jnp.float32),
        scratch_types=(
            [pltpu.VMEM((2, PH * CH), jnp.int32)] * 2 +
            [pltpu.VMEM((PH, CH), jnp.int32)] +
            [pltpu.VMEM((CH, D), jnp.float32)] * NB +
            [pltpu.VMEM_SHARED((N_PAD, D), jnp.float32)] +
            [pltpu.SemaphoreType.DMA] * (NB + 3)
        ),
    )
    def agg(x_hbm, e_hbm, out_hbm,
            eb0, eb1, didx2d, *rest):
        rows = list(rest[:NB])
        acc = rest[NB]
        gsem = list(rest[NB + 1:2 * NB + 1])
        zsem, isem0, isem1 = rest[2 * NB + 1:]
        cid = lax.axis_index("c")
        sid = lax.axis_index("s")
        wid = sid * NC + cid
        rbase = sid * ROWS_PER_TILE

        zval = jnp.zeros((16,), jnp.float32)

        def zrow(i, carry):
            for j in range(D // 16):
                rows[0][i, pl.ds(j * 16, 16)] = zval
            return carry

        lax.fori_loop(0, CH, zrow, 0)
        zcopies = [pltpu.async_copy(rows[0], acc.at[pl.ds(rbase + k * CH, CH)], zsem)
                   for k in range(ROWS_PER_TILE // CH)]
        ztail = ROWS_PER_TILE % CH
        if ztail:
            zcopies.append(pltpu.async_copy(
                rows[0].at[pl.ds(0, ztail)],
                acc.at[pl.ds(rbase + (ROWS_PER_TILE // CH) * CH, ztail)], zsem))

        def repack(ebuf):
            def rp(k, carry):
                didx2d[k // (CH // 16), pl.ds((k % (CH // 16)) * 16, 16)] = (
                    ebuf[1, pl.ds(k * 16, 16)])
                return carry

            lax.fori_loop(0, PH * CH // 16, rp, 0)

        ebase = wid * EPW
        pblk = PH * CH
        ebufs = [(eb0, isem0), (eb1, isem1)]
        pltpu.sync_copy(e_hbm.at[:, pl.ds(ebase, pblk)], eb0)
        pending = {1: pltpu.async_copy(
            e_hbm.at[:, pl.ds(ebase + pblk, pblk)], eb1, isem1)}
        for b in range(1, NB):
            pltpu.async_copy(
                x_hbm.at[eb0.at[0, pl.ds(b * CH, CH)]], rows[b], gsem[b])
        repack(eb0)
        for h in zcopies:
            h.wait()
        pltpu.async_copy(x_hbm.at[eb0.at[0, pl.ds(0, CH)]], rows[0], gsem[0])
        plsc.subcore_barrier()

        for p in range(NPH):
            eb, _ = ebufs[p % 2]
            if 1 <= p and p + 1 < NPH:
                ne, nsem = ebufs[(p + 1) % 2]
                off = ebase + (p + 1) * pblk
                pending[p + 1] = pltpu.async_copy(
                    e_hbm.at[:, pl.ds(off, pblk)], ne, nsem)
            if p in pending:
                pending.pop(p).wait()
            if p > 0:
                for b in range(NB):
                    pltpu.async_copy(
                        x_hbm.at[eb.at[0, pl.ds(b * CH, CH)]], rows[b], gsem[b])
                repack(eb)

            def body(j4, carry, eb=eb):
                a = NB * j4
                for b in range(NB):
                    pltpu.make_async_copy(
                        x_hbm.at[eb.at[0, pl.ds((a + b) * CH, CH)]],
                        rows[b], gsem[b]).wait()
                    pltpu.sync_copy(rows[b], acc.at[didx2d.at[a + b]], add=True)

                    @pl.when(a + b + NB < PH)
                    def _(b=b, a=a, eb=eb):
                        pltpu.async_copy(
                            x_hbm.at[eb.at[0, pl.ds((a + b + NB) * CH, CH)]],
                            rows[b], gsem[b])
                return carry

            lax.fori_loop(0, PH // NB, body, 0)
        plsc.subcore_barrier()

        pltpu.sync_copy(acc.at[pl.ds(rbase, ROWS_PER_TILE)],
                        out_hbm.at[cid, pl.ds(rbase, ROWS_PER_TILE)])

    return agg(x, edges)


def _tc_combine(partials, W, bias):
    BR = 1000

    def body(p_ref, w_ref, b_ref, o_ref):
        s = p_ref[0] + p_ref[1]
        o_ref[...] = lax.dot_general(
            s, w_ref[...], (((1,), (1,)), ((), ())),
            preferred_element_type=jnp.float32) + b_ref[...]

    return pl.pallas_call(
        body,
        grid=(N_NODES // BR,),
        in_specs=[
            pl.BlockSpec((NC, BR, D), lambda i: (0, i, 0)),
            pl.BlockSpec((D, D), lambda i: (0, 0)),
            pl.BlockSpec((1, D), lambda i: (0, 0)),
        ],
        out_specs=pl.BlockSpec((BR, D), lambda i: (i, 0)),
        out_shape=jax.ShapeDtypeStruct((N_NODES, D), jnp.float32),
    )(partials, W, bias.reshape(1, D))


def kernel(x, edge_index, W, bias):
    ei = edge_index.astype(jnp.int32)
    n_extra = E_PAD - N_EDGES
    pad_iota = jnp.arange(n_extra, dtype=jnp.int32)
    pad = jnp.stack([pad_iota % N_NODES,
                     N_NODES + pad_iota % (N_PAD - N_NODES)])
    edges = jnp.concatenate([ei, pad], axis=1)
    partials = _sc_aggregate(x, edges)
    return _tc_combine(partials, W, bias)


# speedup vs baseline: 1.0753x; 1.0348x over previous
import functools

import jax
import jax.numpy as jnp
import numpy as np
from jax import lax
from jax.experimental import pallas as pl
from jax.experimental.pallas import tpu as pltpu
from jax.experimental.pallas import tpu_sc as plsc

N_NODES = 10000
N_PAD = 10112
N_EDGES = 320000
D = 128

NC = 2
NS = 16
NW = NC * NS
CH = 64
NB = 4
NPH = 10
PH = 16
EPW = NPH * PH * CH
E_PAD = NW * EPW
ROWS_PER_TILE = N_PAD // NS

_PAD_IOTA = np.arange(E_PAD - N_EDGES, dtype=np.int32)
_PAD_EDGES = np.stack([_PAD_IOTA % N_NODES,
                       N_NODES + _PAD_IOTA % (N_PAD - N_NODES)])


def _sc_aggregate(x, edges):
    mesh = plsc.VectorSubcoreMesh(core_axis_name="c", subcore_axis_name="s")

    @functools.partial(
        pl.kernel,
        mesh=mesh,
        out_type=jax.ShapeDtypeStruct((NC, N_PAD, D), jnp.float32),
        scratch_types=(
            [pltpu.VMEM((2, PH * CH), jnp.int32)] * 2 +
            [pltpu.VMEM((PH, CH), jnp.int32)] +
            [pltpu.VMEM((CH, D), jnp.float32)] * NB +
            [pltpu.VMEM_SHARED((N_PAD, D), jnp.float32)] +
            [pltpu.SemaphoreType.DMA] * (NB + 3)
        ),
    )
    def agg(x_hbm, e_hbm, out_hbm,
            eb0, eb1, didx2d, *rest):
        rows = list(rest[:NB])
        acc = rest[NB]
        gsem = list(rest[NB + 1:2 * NB + 1])
        zsem, isem0, isem1 = rest[2 * NB + 1:]
        cid = lax.axis_index("c")
        sid = lax.axis_index("s")
        wid = sid * NC + cid
        rbase = sid * ROWS_PER_TILE

        zval = jnp.zeros((16,), jnp.float32)

        def zrow(i, carry):
            for j in range(D // 16):
                rows[0][i, pl.ds(j * 16, 16)] = zval
            return carry

        lax.fori_loop(0, CH, zrow, 0)
        zcopies = [pltpu.async_copy(rows[0], acc.at[pl.ds(rbase + k * CH, CH)], zsem)
                   for k in range(ROWS_PER_TILE // CH)]
        ztail = ROWS_PER_TILE % CH
        if ztail:
            zcopies.append(pltpu.async_copy(
                rows[0].at[pl.ds(0, ztail)],
                acc.at[pl.ds(rbase + (ROWS_PER_TILE // CH) * CH, ztail)], zsem))

        def repack(ebuf):
            def rp(k, carry):
                didx2d[k // (CH // 16), pl.ds((k % (CH // 16)) * 16, 16)] = (
                    ebuf[1, pl.ds(k * 16, 16)])
                return carry

            lax.fori_loop(0, PH * CH // 16, rp, 0)

        ebase = wid * EPW
        pblk = PH * CH
        ebufs = [(eb0, isem0), (eb1, isem1)]
        pltpu.sync_copy(e_hbm.at[:, pl.ds(ebase, pblk)], eb0)
        pending = {1: pltpu.async_copy(
            e_hbm.at[:, pl.ds(ebase + pblk, pblk)], eb1, isem1)}
        for b in range(1, NB):
            pltpu.async_copy(
                x_hbm.at[eb0.at[0, pl.ds(b * CH, CH)]], rows[b], gsem[b])
        repack(eb0)
        for h in zcopies:
            h.wait()
        pltpu.async_copy(x_hbm.at[eb0.at[0, pl.ds(0, CH)]], rows[0], gsem[0])
        plsc.subcore_barrier()

        for p in range(NPH):
            eb, _ = ebufs[p % 2]
            if 1 <= p and p + 1 < NPH:
                ne, nsem = ebufs[(p + 1) % 2]
                off = ebase + (p + 1) * pblk
                pending[p + 1] = pltpu.async_copy(
                    e_hbm.at[:, pl.ds(off, pblk)], ne, nsem)
            if p in pending:
                pending.pop(p).wait()
            if p > 0:
                for b in range(NB):
                    pltpu.async_copy(
                        x_hbm.at[eb.at[0, pl.ds(b * CH, CH)]], rows[b], gsem[b])
                repack(eb)

            def body(j4, carry, eb=eb):
                a = NB * j4
                for b in range(NB):
                    pltpu.make_async_copy(
                        x_hbm.at[eb.at[0, pl.ds((a + b) * CH, CH)]],
                        rows[b], gsem[b]).wait()
                    pltpu.sync_copy(rows[b], acc.at[didx2d.at[a + b]], add=True)

                    @pl.when(a + b + NB < PH)
                    def _(b=b, a=a, eb=eb):
                        pltpu.async_copy(
                            x_hbm.at[eb.at[0, pl.ds((a + b + NB) * CH, CH)]],
                            rows[b], gsem[b])
                return carry

            lax.fori_loop(0, PH // NB, body, 0)
        plsc.subcore_barrier()

        pltpu.sync_copy(acc.at[pl.ds(rbase, ROWS_PER_TILE)],
                        out_hbm.at[cid, pl.ds(rbase, ROWS_PER_TILE)])

    return agg(x, edges)


def _tc_combine(partials, W, bias):
    BR = 2000

    def body(p_ref, w_ref, b_ref, o_ref):
        s = p_ref[0] + p_ref[1]
        o_ref[...] = lax.dot_general(
            s, w_ref[...], (((1,), (1,)), ((), ())),
            preferred_element_type=jnp.float32) + b_ref[...]

    return pl.pallas_call(
        body,
        grid=(N_NODES // BR,),
        in_specs=[
            pl.BlockSpec((NC, BR, D), lambda i: (0, i, 0)),
            pl.BlockSpec((D, D), lambda i: (0, 0)),
            pl.BlockSpec((1, D), lambda i: (0, 0)),
        ],
        out_specs=pl.BlockSpec((BR, D), lambda i: (i, 0)),
        out_shape=jax.ShapeDtypeStruct((N_NODES, D), jnp.float32),
    )(partials, W, bias.reshape(1, D))


def kernel(x, edge_index, W, bias):
    ei = edge_index.astype(jnp.int32)
    edges = jnp.concatenate([ei, jnp.asarray(_PAD_EDGES)], axis=1)
    partials = _sc_aggregate(x, edges)
    return _tc_combine(partials, W, bias)


# speedup vs baseline: 1.0980x; 1.0211x over previous
import functools

import jax
import jax.numpy as jnp
import numpy as np
from jax import lax
from jax.experimental import pallas as pl
from jax.experimental.pallas import tpu as pltpu
from jax.experimental.pallas import tpu_sc as plsc

N_NODES = 10000
N_PAD = 10112
N_EDGES = 320000
D = 128

NC = 2
NS = 16
NW = NC * NS
CH = 64
NB = 4
NPH = 8
PH = 20
EPW = NPH * PH * CH
E_PAD = NW * EPW
ROWS_PER_TILE = N_PAD // NS

_PAD_IOTA = np.arange(E_PAD - N_EDGES, dtype=np.int32)
_PAD_EDGES = np.stack([_PAD_IOTA % N_NODES,
                       N_NODES + _PAD_IOTA % (N_PAD - N_NODES)])


def _sc_aggregate(x, edges):
    mesh = plsc.VectorSubcoreMesh(core_axis_name="c", subcore_axis_name="s")

    @functools.partial(
        pl.kernel,
        mesh=mesh,
        out_type=jax.ShapeDtypeStruct((NC, N_PAD, D), jnp.float32),
        scratch_types=(
            [pltpu.VMEM((2, PH * CH), jnp.int32)] * 2 +
            [pltpu.VMEM((PH, CH), jnp.int32)] +
            [pltpu.VMEM((CH, D), jnp.float32)] * NB +
            [pltpu.VMEM_SHARED((N_PAD, D), jnp.float32)] +
            [pltpu.SemaphoreType.DMA] * (NB + 3)
        ),
    )
    def agg(x_hbm, e_hbm, out_hbm,
            eb0, eb1, didx2d, *rest):
        rows = list(rest[:NB])
        acc = rest[NB]
        gsem = list(rest[NB + 1:2 * NB + 1])
        zsem, isem0, isem1 = rest[2 * NB + 1:]
        cid = lax.axis_index("c")
        sid = lax.axis_index("s")
        wid = sid * NC + cid
        rbase = sid * ROWS_PER_TILE

        zval = jnp.zeros((16,), jnp.float32)

        def zrow(i, carry):
            for j in range(D // 16):
                rows[0][i, pl.ds(j * 16, 16)] = zval
            return carry

        lax.fori_loop(0, CH, zrow, 0)
        zcopies = [pltpu.async_copy(rows[0], acc.at[pl.ds(rbase + k * CH, CH)], zsem)
                   for k in range(ROWS_PER_TILE // CH)]
        ztail = ROWS_PER_TILE % CH
        if ztail:
            zcopies.append(pltpu.async_copy(
                rows[0].at[pl.ds(0, ztail)],
                acc.at[pl.ds(rbase + (ROWS_PER_TILE // CH) * CH, ztail)], zsem))

        def repack(ebuf):
            def rp(k, carry):
                didx2d[k // (CH // 16), pl.ds((k % (CH // 16)) * 16, 16)] = (
                    ebuf[1, pl.ds(k * 16, 16)])
                return carry

            lax.fori_loop(0, PH * CH // 16, rp, 0)

        ebase = wid * EPW
        pblk = PH * CH
        ebufs = [(eb0, isem0), (eb1, isem1)]
        pltpu.sync_copy(e_hbm.at[:, pl.ds(ebase, pblk)], eb0)
        pending = {1: pltpu.async_copy(
            e_hbm.at[:, pl.ds(ebase + pblk, pblk)], eb1, isem1)}
        for b in range(1, NB):
            pltpu.async_copy(
                x_hbm.at[eb0.at[0, pl.ds(b * CH, CH)]], rows[b], gsem[b])
        repack(eb0)
        for h in zcopies:
            h.wait()
        pltpu.async_copy(x_hbm.at[eb0.at[0, pl.ds(0, CH)]], rows[0], gsem[0])
        plsc.subcore_barrier()

        for p in range(NPH):
            eb, _ = ebufs[p % 2]
            if 1 <= p and p + 1 < NPH:
                ne, nsem = ebufs[(p + 1) % 2]
                off = ebase + (p + 1) * pblk
                pending[p + 1] = pltpu.async_copy(
                    e_hbm.at[:, pl.ds(off, pblk)], ne, nsem)
            if p in pending:
                pending.pop(p).wait()
            if p > 0:
                for b in range(NB):
                    pltpu.async_copy(
                        x_hbm.at[eb.at[0, pl.ds(b * CH, CH)]], rows[b], gsem[b])
                repack(eb)

            def body(j4, carry, eb=eb):
                a = NB * j4
                for b in range(NB):
                    pltpu.make_async_copy(
                        x_hbm.at[eb.at[0, pl.ds((a + b) * CH, CH)]],
                        rows[b], gsem[b]).wait()
                    pltpu.sync_copy(rows[b], acc.at[didx2d.at[a + b]], add=True)

                    @pl.when(a + b + NB < PH)
                    def _(b=b, a=a, eb=eb):
                        pltpu.async_copy(
                            x_hbm.at[eb.at[0, pl.ds((a + b + NB) * CH, CH)]],
                            rows[b], gsem[b])
                return carry

            lax.fori_loop(0, PH // NB, body, 0)
        plsc.subcore_barrier()

        pltpu.sync_copy(acc.at[pl.ds(rbase, ROWS_PER_TILE)],
                        out_hbm.at[cid, pl.ds(rbase, ROWS_PER_TILE)])

    return agg(x, edges)


def _tc_combine(partials, W, bias):
    BR = 2000

    def body(p_ref, w_ref, b_ref, o_ref):
        s = p_ref[0] + p_ref[1]
        o_ref[...] = lax.dot_general(
            s, w_ref[...], (((1,), (1,)), ((), ())),
            preferred_element_type=jnp.float32) + b_ref[...]

    return pl.pallas_call(
        body,
        grid=(N_NODES // BR,),
        in_specs=[
            pl.BlockSpec((NC, BR, D), lambda i: (0, i, 0)),
            pl.BlockSpec((D, D), lambda i: (0, 0)),
            pl.BlockSpec((1, D), lambda i: (0, 0)),
        ],
        out_specs=pl.BlockSpec((BR, D), lambda i: (i, 0)),
        out_shape=jax.ShapeDtypeStruct((N_NODES, D), jnp.float32),
    )(partials, W, bias.reshape(1, D))


def kernel(x, edge_index, W, bias):
    ei = edge_index.astype(jnp.int32)
    edges = jnp.concatenate([ei, jnp.asarray(_PAD_EDGES)], axis=1)
    partials = _sc_aggregate(x, edges)
    return _tc_combine(partials, W, bias)
